# baseline (speedup 1.0000x reference)
import functools

import jax
import jax.numpy as jnp
from jax import lax
from jax.experimental import pallas as pl
from jax.experimental.pallas import tpu as pltpu
from jax.experimental.pallas import tpu_sc as plsc

_BS, _C1, _NY, _NX = 4, 96, 224, 224
_C2 = 96
_CP = 128
_K = 3
_PY, _PX = _NY + 2, _NX + 2
_NW = 32
_CHUNK = 128
_BN = 512


_D = 4


def _sc_gather_build(n_pad):
    npw = n_pad // _NW
    nch = npw // _CHUNK
    ngrp = nch // _D
    ntail = nch % _D
    mesh = plsc.VectorSubcoreMesh(core_axis_name="c", subcore_axis_name="s")

    @functools.partial(
        pl.kernel,
        mesh=mesh,
        out_type=jax.ShapeDtypeStruct((_K * _K, n_pad, _CP), jnp.float32),
        scratch_types=[
            pltpu.VMEM((npw,), jnp.int32),
            pltpu.VMEM((npw,), jnp.int32),
            pltpu.VMEM((npw,), jnp.int32),
            pltpu.VMEM((npw,), jnp.int32),
            [pltpu.VMEM((_CHUNK,), jnp.int32) for _ in range(_D)],
            [pltpu.VMEM((_CHUNK, _CP), jnp.float32) for _ in range(_D)],
            [pltpu.SemaphoreType.DMA for _ in range(_D)],
            pltpu.SemaphoreType.DMA,
        ],
    )
    def sc_gather(table_hbm, bi_hbm, yi_hbm, xi_hbm, out_hbm,
                  bi_v, yi_v, xi_v, base_v, idx_bufs, rows_bufs,
                  sem_g, sem_w):
        ci = lax.axis_index("c")
        si = lax.axis_index("s")
        wid = si * 2 + ci
        pbase = wid * npw
        pltpu.sync_copy(bi_hbm.at[pl.ds(pbase, npw)], bi_v)
        pltpu.sync_copy(yi_hbm.at[pl.ds(pbase, npw)], yi_v)
        pltpu.sync_copy(xi_hbm.at[pl.ds(pbase, npw)], xi_v)

        def calc_base(j, carry):
            s = j * 16
            b16 = bi_v[pl.ds(s, 16)]
            y16 = yi_v[pl.ds(s, 16)]
            x16 = xi_v[pl.ds(s, 16)]
            base_v[pl.ds(s, 16)] = (b16 * _PY + y16) * _PX + x16
            return carry

        lax.fori_loop(0, npw // 16, calc_base, 0)

        def fill_idx(idx_v, c, off):
            def calc_idx(j, carry2):
                s = j * 16
                idx_v[pl.ds(s, 16)] = base_v[pl.ds(c * _CHUNK + s, 16)] + off
                return carry2

            lax.fori_loop(0, _CHUNK // 16, calc_idx, 0)

        def run_group(c0, off, t, width):
            gs = []
            for k in range(width):
                fill_idx(idx_bufs[k], c0 + k, off)
                gs.append(pltpu.async_copy(
                    table_hbm.at[idx_bufs[k]], rows_bufs[k], sem_g[k]))
            ws = []
            for k in range(width):
                gs[k].wait()
                ws.append(pltpu.async_copy(
                    rows_bufs[k],
                    out_hbm.at[t, pl.ds(pbase + (c0 + k) * _CHUNK, _CHUNK)],
                    sem_w))
            for w in ws:
                w.wait()

        for t in range(_K * _K):
            off = (t // _K) * _PX + (t % _K)

            def per_group(i, carry):
                run_group(i * _D, off, t, _D)
                return carry

            lax.fori_loop(0, ngrp, per_group, 0)
            if ntail:
                run_group(ngrp * _D, off, t, ntail)

    return sc_gather


def _gemm_body(g_ref, w_ref, b_ref, o_ref):
    acc = jnp.broadcast_to(b_ref[...], (o_ref.shape[0], _C2))
    for t in range(_K * _K):
        acc = acc + jnp.dot(g_ref[t], w_ref[t],
                            preferred_element_type=jnp.float32)
    o_ref[...] = acc


def _tc_gemm(g3, w3, bias_row):
    n_pad = g3.shape[1]
    nt = _K * _K
    return pl.pallas_call(
        _gemm_body,
        grid=(n_pad // _BN,),
        in_specs=[
            pl.BlockSpec((nt, _BN, _CP), lambda i: (0, i, 0)),
            pl.BlockSpec((nt, _CP, _C2), lambda i: (0, 0, 0)),
            pl.BlockSpec((1, _C2), lambda i: (0, 0)),
        ],
        out_specs=pl.BlockSpec((_BN, _C2), lambda i: (i, 0)),
        out_shape=jax.ShapeDtypeStruct((n_pad, _C2), jnp.float32),
    )(g3, w3, bias_row)


def kernel(x, indices, weight_flatten, bias):
    n = indices.shape[0]
    n_pad = ((n + _NW * _CHUNK - 1) // (_NW * _CHUNK)) * (_NW * _CHUNK)

    xt = jnp.pad(jnp.transpose(x, (0, 2, 3, 1)),
                 ((0, 0), (1, 1), (1, 1), (0, _CP - _C1)))
    table = xt.reshape(_BS * _PY * _PX, _CP)

    idx = indices.astype(jnp.int32)
    bi = jnp.pad(idx[:, 0], (0, n_pad - n))
    yi = jnp.pad(idx[:, 1], (0, n_pad - n))
    xi = jnp.pad(idx[:, 2], (0, n_pad - n))

    g3 = _sc_gather_build(n_pad)(table, bi, yi, xi)

    w3 = weight_flatten.reshape(_C2, _C1, _K * _K).transpose(2, 1, 0)
    w3 = jnp.pad(w3, ((0, 0), (0, _CP - _C1), (0, 0)))

    z = _tc_gemm(g3, w3, bias.reshape(1, _C2))
    return z[:n]
